# baseline (speedup 1.0000x reference)
import jax
import jax.numpy as jnp
from jax.experimental import pallas as pl
from jax.experimental.pallas import tpu as pltpu
from jax.experimental.pallas import tpu_sc as plsc

_GATHER_WINDOW = 128
_BV = 2048


def _sc_gather_packed(packed_table, packed_idx):
    n = packed_idx.shape[0]
    idx2d = packed_idx.reshape(1, n)
    mesh = plsc.VectorSubcoreMesh(core_axis_name="core", subcore_axis_name="subcore")

    @pl.kernel(
        out_type=jax.ShapeDtypeStruct((n, packed_table.shape[1]), packed_table.dtype),
        mesh=mesh,
    )
    def gather_kernel(table_hbm, idx_hbm, out_hbm):
        def body(idx_vmem, out_vmem):
            pltpu.sync_copy(table_hbm.at[idx_vmem.at[0]], out_vmem)

        pltpu.emit_pipeline(
            body,
            grid=(n // _GATHER_WINDOW,),
            in_specs=[pl.BlockSpec((1, _GATHER_WINDOW), index_map=lambda i: (0, i))],
            out_specs=[
                pl.BlockSpec(
                    (_GATHER_WINDOW, packed_table.shape[1]),
                    index_map=lambda i: (i, 0),
                )
            ],
            core_axis_name="subcore",
            dimension_semantics=(pltpu.PARALLEL,),
        )(idx_hbm, out_hbm)

    return gather_kernel(packed_table, idx2d)


def _tc_project(h_packed, parity, W, b2d):
    batch = h_packed.shape[0]
    dim, vocab = W.shape
    grid = pl.cdiv(vocab, _BV)

    def mm_kernel(hp_ref, par_ref, w_ref, b_ref, o_ref):
        h = jnp.where(par_ref[...] != 0, hp_ref[:, dim:], hp_ref[:, :dim])
        o_ref[...] = (
            jnp.dot(h, w_ref[...], preferred_element_type=jnp.float32) + b_ref[...]
        )

    return pl.pallas_call(
        mm_kernel,
        grid=(grid,),
        in_specs=[
            pl.BlockSpec((batch, 2 * dim), lambda j: (0, 0)),
            pl.BlockSpec((batch, 1), lambda j: (0, 0)),
            pl.BlockSpec((dim, _BV), lambda j: (0, j)),
            pl.BlockSpec((1, _BV), lambda j: (0, j)),
        ],
        out_specs=pl.BlockSpec((batch, _BV), lambda j: (0, j)),
        out_shape=jax.ShapeDtypeStruct((batch, vocab), jnp.float32),
        compiler_params=pltpu.CompilerParams(
            dimension_semantics=("parallel",),
        ),
    )(h_packed, parity, W, b2d)


def kernel(input_ids, embed_table, W, b):
    vocab_rows, dim = embed_table.shape
    packed_table = embed_table.reshape(vocab_rows // 2, 2 * dim)
    h_packed = _sc_gather_packed(packed_table, input_ids // 2)
    parity = (input_ids % 2).astype(jnp.int32).reshape(-1, 1)
    return _tc_project(h_packed, parity, W, b.reshape(1, -1))


# speedup vs baseline: 1.0087x; 1.0087x over previous
import jax
import jax.numpy as jnp
from jax.experimental import pallas as pl
from jax.experimental.pallas import tpu as pltpu
from jax.experimental.pallas import tpu_sc as plsc

_GATHER_WINDOW = 128
_BV = 4096


def _sc_gather_packed(packed_table, packed_idx):
    n = packed_idx.shape[0]
    idx2d = packed_idx.reshape(1, n)
    mesh = plsc.VectorSubcoreMesh(core_axis_name="core", subcore_axis_name="subcore")

    @pl.kernel(
        out_type=jax.ShapeDtypeStruct((n, packed_table.shape[1]), packed_table.dtype),
        mesh=mesh,
    )
    def gather_kernel(table_hbm, idx_hbm, out_hbm):
        def body(idx_vmem, out_vmem):
            pltpu.sync_copy(table_hbm.at[idx_vmem.at[0]], out_vmem)

        pltpu.emit_pipeline(
            body,
            grid=(n // _GATHER_WINDOW,),
            in_specs=[pl.BlockSpec((1, _GATHER_WINDOW), index_map=lambda i: (0, i))],
            out_specs=[
                pl.BlockSpec(
                    (_GATHER_WINDOW, packed_table.shape[1]),
                    index_map=lambda i: (i, 0),
                )
            ],
            core_axis_name="subcore",
            dimension_semantics=(pltpu.PARALLEL,),
        )(idx_hbm, out_hbm)

    return gather_kernel(packed_table, idx2d)


def _tc_project(h_packed, parity, W, b2d):
    batch = h_packed.shape[0]
    dim, vocab = W.shape
    grid = pl.cdiv(vocab, _BV)

    def mm_kernel(hp_ref, par_ref, w_ref, b_ref, o_ref):
        h = jnp.where(par_ref[...] != 0, hp_ref[:, dim:], hp_ref[:, :dim])
        o_ref[...] = (
            jnp.dot(h, w_ref[...], preferred_element_type=jnp.float32) + b_ref[...]
        )

    return pl.pallas_call(
        mm_kernel,
        grid=(grid,),
        in_specs=[
            pl.BlockSpec((batch, 2 * dim), lambda j: (0, 0)),
            pl.BlockSpec((batch, 1), lambda j: (0, 0)),
            pl.BlockSpec((dim, _BV), lambda j: (0, j)),
            pl.BlockSpec((1, _BV), lambda j: (0, j)),
        ],
        out_specs=pl.BlockSpec((batch, _BV), lambda j: (0, j)),
        out_shape=jax.ShapeDtypeStruct((batch, vocab), jnp.float32),
        compiler_params=pltpu.CompilerParams(
            dimension_semantics=("parallel",),
        ),
    )(h_packed, parity, W, b2d)


def kernel(input_ids, embed_table, W, b):
    vocab_rows, dim = embed_table.shape
    packed_table = embed_table.reshape(vocab_rows // 2, 2 * dim)
    h_packed = _sc_gather_packed(packed_table, input_ids // 2)
    parity = (input_ids % 2).astype(jnp.int32).reshape(-1, 1)
    return _tc_project(h_packed, parity, W, b.reshape(1, -1))
